# baseline (speedup 1.0000x reference)
import functools
import jax
import jax.numpy as jnp
from jax import lax
from jax.experimental import pallas as pl
from jax.experimental.pallas import tpu as pltpu
from jax.experimental.pallas import tpu_sc as plsc

_NW = 32
_R = 8
_D = 1024
_CHUNK = _R * _D


def _sc_body(feat_hbm, pos_hbm, out_hbm, pos_v, feat_v, B, S):
    wid = lax.axis_index("c") * 16 + lax.axis_index("s")
    s_per_w = S // _NW
    n_chunks = s_per_w // _R
    s0 = wid * s_per_w

    def chunk(c, _):
        row = s0 + c * _R
        pltpu.sync_copy(pos_hbm.at[pl.ds(row * _D, _CHUNK)], pos_v)
        for b in range(B):
            fbase = (b * S + row) * _D
            pltpu.sync_copy(feat_hbm.at[pl.ds(fbase, _CHUNK)], feat_v.at[b])

            def add16(i, _):
                off = i * 16
                v = pos_v[pl.ds(off, 16)]
                plsc.addupdate(feat_v.at[b, pl.ds(off, 16)], v)
                return 0

            lax.fori_loop(0, _CHUNK // 16, add16, 0)
            pltpu.sync_copy(feat_v.at[b], out_hbm.at[pl.ds(fbase, _CHUNK)])
        return 0

    lax.fori_loop(0, n_chunks, chunk, 0)


def kernel(feat, pos_emb):
    B, S, D = feat.shape
    pe = pos_emb[:S].reshape(S * D)
    feat_flat = feat.reshape(B * S * D)
    mesh = plsc.VectorSubcoreMesh(core_axis_name="c", subcore_axis_name="s")
    out = pl.kernel(
        functools.partial(_sc_body, B=B, S=S),
        out_type=jax.ShapeDtypeStruct((B * S * D,), feat.dtype),
        mesh=mesh,
        scratch_types=[
            pltpu.VMEM((_CHUNK,), jnp.float32),
            pltpu.VMEM((4, _CHUNK), jnp.float32),
        ],
    )(feat_flat, pe)
    return out.reshape(B, S, D)


# speedup vs baseline: 1.3332x; 1.3332x over previous
import functools
import jax
import jax.numpy as jnp
from jax import lax
from jax.experimental import pallas as pl
from jax.experimental.pallas import tpu as pltpu
from jax.experimental.pallas import tpu_sc as plsc

_NW = 32
_R = 8
_D = 1024
_CHUNK = _R * _D


def _sc_body(feat_hbm, pos_hbm, out_hbm, pos_v, feat_v,
             in_sem0, in_sem1, out_sem0, out_sem1, B, S):
    wid = lax.axis_index("c") * 16 + lax.axis_index("s")
    s_per_w = S // _NW
    n_chunks = s_per_w // _R
    s0 = wid * s_per_w
    in_sems = (in_sem0, in_sem1)
    out_sems = (out_sem0, out_sem1)

    def in_copies(c, slot):
        row = s0 + c * _R
        copies = [pltpu.make_async_copy(
            pos_hbm.at[pl.ds(row * _D, _CHUNK)], pos_v.at[slot],
            in_sems[slot])]
        for b in range(B):
            fbase = (b * S + row) * _D
            copies.append(pltpu.make_async_copy(
                feat_hbm.at[pl.ds(fbase, _CHUNK)], feat_v.at[slot, b],
                in_sems[slot]))
        return copies

    def out_copy(c, slot, b):
        row = s0 + c * _R
        fbase = (b * S + row) * _D
        return pltpu.make_async_copy(
            feat_v.at[slot, b], out_hbm.at[pl.ds(fbase, _CHUNK)],
            out_sems[slot])

    def issue_loads(c, slot):
        for cp in in_copies(c, slot):
            cp.start()

    def wait_stores(c, slot):
        for b in range(B):
            out_copy(c, slot, b).wait()

    def compute_store(c, slot):
        for cp in in_copies(c, slot):
            cp.wait()
        for b in range(B):
            def add16(i, _):
                off = i * 16
                v = pos_v[slot, pl.ds(off, 16)]
                plsc.addupdate(feat_v.at[slot, b, pl.ds(off, 16)], v)
                return 0

            lax.fori_loop(0, _CHUNK // 16, add16, 0)
            out_copy(c, slot, b).start()

    issue_loads(0, 0)

    def body2(t, _):
        c0 = 2 * t

        @pl.when(t > 0)
        def _():
            wait_stores(c0 - 1, 1)

        issue_loads(c0 + 1, 1)
        compute_store(c0, 0)
        compute_store(c0 + 1, 1)

        @pl.when(c0 + 2 < n_chunks)
        def _():
            wait_stores(c0, 0)
            issue_loads(c0 + 2, 0)
        return 0

    lax.fori_loop(0, n_chunks // 2, body2, 0)
    wait_stores(n_chunks - 2, 0)
    wait_stores(n_chunks - 1, 1)


def kernel(feat, pos_emb):
    B, S, D = feat.shape
    pe = pos_emb[:S].reshape(S * D)
    feat_flat = feat.reshape(B * S * D)
    mesh = plsc.VectorSubcoreMesh(core_axis_name="c", subcore_axis_name="s")
    out = pl.kernel(
        functools.partial(_sc_body, B=B, S=S),
        out_type=jax.ShapeDtypeStruct((B * S * D,), feat.dtype),
        mesh=mesh,
        scratch_types=[
            pltpu.VMEM((2, _CHUNK), jnp.float32),
            pltpu.VMEM((2, 4, _CHUNK), jnp.float32),
            pltpu.SemaphoreType.DMA,
            pltpu.SemaphoreType.DMA,
            pltpu.SemaphoreType.DMA,
            pltpu.SemaphoreType.DMA,
        ],
    )(feat_flat, pe)
    return out.reshape(B, S, D)


# speedup vs baseline: 1.3569x; 1.0177x over previous
import functools
import jax
import jax.numpy as jnp
from jax import lax
from jax.experimental import pallas as pl
from jax.experimental.pallas import tpu as pltpu
from jax.experimental.pallas import tpu_sc as plsc

_NW = 32
_R = 8
_D = 1024
_CHUNK = _R * _D
_U = 16


def _sc_body(feat_hbm, pos_hbm, out_hbm, pos_v, feat_v,
             in_sem0, in_sem1, out_sem0, out_sem1, B, S):
    wid = lax.axis_index("c") * 16 + lax.axis_index("s")
    s_per_w = S // _NW
    n_chunks = s_per_w // _R
    s0 = wid * s_per_w
    in_sems = (in_sem0, in_sem1)
    out_sems = (out_sem0, out_sem1)

    def in_copies(c, slot):
        row = s0 + c * _R
        copies = [pltpu.make_async_copy(
            pos_hbm.at[pl.ds(row * _D, _CHUNK)], pos_v.at[slot],
            in_sems[slot])]
        for b in range(B):
            fbase = (b * S + row) * _D
            copies.append(pltpu.make_async_copy(
                feat_hbm.at[pl.ds(fbase, _CHUNK)], feat_v.at[slot, b],
                in_sems[slot]))
        return copies

    def out_copy(c, slot, b):
        row = s0 + c * _R
        fbase = (b * S + row) * _D
        return pltpu.make_async_copy(
            feat_v.at[slot, b], out_hbm.at[pl.ds(fbase, _CHUNK)],
            out_sems[slot])

    def issue_loads(c, slot):
        for cp in in_copies(c, slot):
            cp.start()

    def wait_stores(c, slot):
        for b in range(B):
            out_copy(c, slot, b).wait()

    def compute_store(c, slot):
        for cp in in_copies(c, slot):
            cp.wait()
        for b in range(B):
            def add16(i, _):
                base = i * (16 * _U)
                for u in range(_U):
                    off = base + u * 16
                    v = pos_v[slot, pl.ds(off, 16)]
                    plsc.addupdate(feat_v.at[slot, b, pl.ds(off, 16)], v)
                return 0

            lax.fori_loop(0, _CHUNK // (16 * _U), add16, 0)
            out_copy(c, slot, b).start()

    issue_loads(0, 0)

    def body2(t, _):
        c0 = 2 * t

        @pl.when(t > 0)
        def _():
            wait_stores(c0 - 1, 1)

        issue_loads(c0 + 1, 1)
        compute_store(c0, 0)
        compute_store(c0 + 1, 1)

        @pl.when(c0 + 2 < n_chunks)
        def _():
            wait_stores(c0, 0)
            issue_loads(c0 + 2, 0)
        return 0

    lax.fori_loop(0, n_chunks // 2, body2, 0)
    wait_stores(n_chunks - 2, 0)
    wait_stores(n_chunks - 1, 1)


def kernel(feat, pos_emb):
    B, S, D = feat.shape
    pe = pos_emb[:S].reshape(S * D)
    feat_flat = feat.reshape(B * S * D)
    mesh = plsc.VectorSubcoreMesh(core_axis_name="c", subcore_axis_name="s")
    out = pl.kernel(
        functools.partial(_sc_body, B=B, S=S),
        out_type=jax.ShapeDtypeStruct((B * S * D,), feat.dtype),
        mesh=mesh,
        scratch_types=[
            pltpu.VMEM((2, _CHUNK), jnp.float32),
            pltpu.VMEM((2, 4, _CHUNK), jnp.float32),
            pltpu.SemaphoreType.DMA,
            pltpu.SemaphoreType.DMA,
            pltpu.SemaphoreType.DMA,
            pltpu.SemaphoreType.DMA,
        ],
    )(feat_flat, pe)
    return out.reshape(B, S, D)


# speedup vs baseline: 4.5882x; 3.3814x over previous
import functools
import jax
import jax.numpy as jnp
from jax import lax
from jax.experimental import pallas as pl
from jax.experimental.pallas import tpu as pltpu
from jax.experimental.pallas import tpu_sc as plsc

_NW = 32
_R = 8


def _sc_body(feat_hbm, pos_hbm, out_hbm, pos_v, feat_v,
             in_sem0, in_sem1, out_sem0, out_sem1, B, S, D):
    wid = lax.axis_index("c") * 16 + lax.axis_index("s")
    s_per_w = S // _NW
    n_chunks = s_per_w // _R
    s0 = wid * s_per_w
    in_sems = (in_sem0, in_sem1)
    out_sems = (out_sem0, out_sem1)

    def in_copies(c, slot):
        row = s0 + c * _R
        copies = [pltpu.make_async_copy(
            pos_hbm.at[pl.ds(row, _R)], pos_v.at[slot], in_sems[slot])]
        for b in range(B):
            copies.append(pltpu.make_async_copy(
                feat_hbm.at[b, pl.ds(row, _R)], feat_v.at[slot, b],
                in_sems[slot]))
        return copies

    def out_copy(c, slot, b):
        row = s0 + c * _R
        return pltpu.make_async_copy(
            feat_v.at[slot, b], out_hbm.at[b, pl.ds(row, _R)],
            out_sems[slot])

    def issue_loads(c, slot):
        for cp in in_copies(c, slot):
            cp.start()

    def wait_stores(c, slot):
        for b in range(B):
            out_copy(c, slot, b).wait()

    def compute_store(c, slot):
        for cp in in_copies(c, slot):
            cp.wait()
        for b in range(B):
            def add_row(r, _):
                for u in range(D // 16):
                    off = u * 16
                    v = pos_v[slot, r, pl.ds(off, 16)]
                    plsc.addupdate(feat_v.at[slot, b, r, pl.ds(off, 16)], v)
                return 0

            lax.fori_loop(0, _R, add_row, 0)
            out_copy(c, slot, b).start()

    issue_loads(0, 0)

    def body2(t, _):
        c0 = 2 * t

        @pl.when(t > 0)
        def _():
            wait_stores(c0 - 1, 1)

        issue_loads(c0 + 1, 1)
        compute_store(c0, 0)
        compute_store(c0 + 1, 1)

        @pl.when(c0 + 2 < n_chunks)
        def _():
            wait_stores(c0, 0)
            issue_loads(c0 + 2, 0)
        return 0

    lax.fori_loop(0, n_chunks // 2, body2, 0)
    wait_stores(n_chunks - 2, 0)
    wait_stores(n_chunks - 1, 1)


def kernel(feat, pos_emb):
    B, S, D = feat.shape
    pe = pos_emb[:S]
    mesh = plsc.VectorSubcoreMesh(core_axis_name="c", subcore_axis_name="s")
    return pl.kernel(
        functools.partial(_sc_body, B=B, S=S, D=D),
        out_type=jax.ShapeDtypeStruct((B, S, D), feat.dtype),
        mesh=mesh,
        scratch_types=[
            pltpu.VMEM((2, _R, D), jnp.float32),
            pltpu.VMEM((2, 4, _R, D), jnp.float32),
            pltpu.SemaphoreType.DMA,
            pltpu.SemaphoreType.DMA,
            pltpu.SemaphoreType.DMA,
            pltpu.SemaphoreType.DMA,
        ],
    )(feat, pe)
